# baseline (speedup 1.0000x reference)
import functools
import math

import jax
import jax.numpy as jnp
from jax import lax
from jax.experimental import pallas as pl
from jax.experimental.pallas import tpu as pltpu
from jax.experimental.pallas import tpu_sc as plsc

_NC = 2
_NS = 16
_NW = _NC * _NS
_LANES = 16

_CHUNK = 128


@functools.lru_cache(maxsize=None)
def _build(n_rows, vocab, d_model, scale):
    assert n_rows % (_NW * _CHUNK) == 0
    rows_per_w = n_rows // _NW
    n_chunks = rows_per_w // _CHUNK
    d_regs = d_model // _LANES

    mesh = plsc.VectorSubcoreMesh(core_axis_name="c", subcore_axis_name="s")

    @functools.partial(
        pl.kernel,
        mesh=mesh,
        out_type=jax.ShapeDtypeStruct((n_rows, d_model), jnp.float32),
        scratch_types=[
            pltpu.VMEM((_CHUNK,), jnp.int32),
            pltpu.VMEM((_CHUNK, d_model), jnp.float32),
            pltpu.SemaphoreType.DMA,
        ],
        compiler_params=pltpu.CompilerParams(use_tc_tiling_on_sc=False),
    )
    def emb(x_hbm, table_hbm, out_hbm, idx_v, rows_v, sem):
        wid = lax.axis_index("s") * _NC + lax.axis_index("c")
        base = wid * rows_per_w

        def chunk_body(g, carry):
            off = base + g * _CHUNK
            pltpu.sync_copy(x_hbm.at[pl.ds(off, _CHUNK)], idx_v)
            pltpu.async_copy(table_hbm.at[idx_v], rows_v, sem).wait()

            def scale_body(i, c):
                for j in range(d_regs):
                    sl = pl.ds(j * _LANES, _LANES)
                    rows_v[i, sl] = rows_v[i, sl] * scale
                return c

            lax.fori_loop(0, _CHUNK, scale_body, 0)
            pltpu.sync_copy(rows_v, out_hbm.at[pl.ds(off, _CHUNK)])
            return carry

        lax.fori_loop(0, n_chunks, chunk_body, 0)

    return emb


def kernel(x, table):
    vocab, d_model = table.shape
    xf = x.reshape(-1).astype(jnp.int32)
    scale = float(math.sqrt(d_model))
    out = _build(xf.shape[0], vocab, d_model, scale)(xf, table)
    return out.reshape(x.shape + (d_model,))


# speedup vs baseline: 1.2673x; 1.2673x over previous
import functools
import math

import jax
import jax.numpy as jnp
from jax import lax
from jax.experimental import pallas as pl
from jax.experimental.pallas import tpu as pltpu
from jax.experimental.pallas import tpu_sc as plsc

_NC = 2
_NS = 16
_NW = _NC * _NS
_LANES = 16

_CHUNK = 128
_NBUF = 4


@functools.lru_cache(maxsize=None)
def _build(n_rows, vocab, d_model, scale):
    assert n_rows % (_NW * _CHUNK * _NBUF) == 0
    rows_per_w = n_rows // _NW
    n_chunks = rows_per_w // _CHUNK
    d_regs = d_model // _LANES

    mesh = plsc.VectorSubcoreMesh(core_axis_name="c", subcore_axis_name="s")

    @functools.partial(
        pl.kernel,
        mesh=mesh,
        out_type=jax.ShapeDtypeStruct((n_rows, d_model), jnp.float32),
        scratch_types=[
            pltpu.VMEM((n_chunks, _CHUNK), jnp.int32),
            [pltpu.VMEM((_CHUNK, d_model), jnp.float32) for _ in range(_NBUF)],
            [pltpu.SemaphoreType.DMA for _ in range(_NBUF)],
            [pltpu.SemaphoreType.DMA for _ in range(_NBUF)],
        ],
        compiler_params=pltpu.CompilerParams(use_tc_tiling_on_sc=False),
    )
    def emb(x_hbm, table_hbm, out_hbm, idx_v, rows_v, gsems, osems):
        wid = lax.axis_index("s") * _NC + lax.axis_index("c")
        base = wid * rows_per_w

        pltpu.sync_copy(x_hbm.at[wid], idx_v)

        def gdesc(g, b):
            return pltpu.make_async_copy(
                table_hbm.at[idx_v.at[g]], rows_v[b], gsems[b]
            )

        def wdesc(g, b):
            off = base + g * _CHUNK
            return pltpu.make_async_copy(
                rows_v[b], out_hbm.at[pl.ds(off, _CHUNK)], osems[b]
            )

        def scale_and_emit(g, b):
            gdesc(g, b).wait()

            @plsc.parallel_loop(0, _CHUNK, 1, unroll=8)
            def _(i):
                for j in range(d_regs):
                    sl = pl.ds(j * _LANES, _LANES)
                    rows_v[b][i, sl] = rows_v[b][i, sl] * scale

            wdesc(g, b).start()

        for b in range(_NBUF):
            gdesc(b, b).start()

        def outer(go, carry):
            g0 = go * _NBUF
            for b in range(_NBUF):
                scale_and_emit(g0 + b, b)
            @pl.when(g0 + _NBUF < n_chunks)
            def _():
                for b in range(_NBUF):
                    wdesc(g0 + b, b).wait()
                    gdesc(g0 + _NBUF + b, b).start()

            return carry

        lax.fori_loop(0, n_chunks // _NBUF, outer, 0)

        for b in range(_NBUF):
            wdesc(n_chunks - _NBUF + b, b).wait()

    return emb


def kernel(x, table):
    vocab, d_model = table.shape
    n_rows = x.size
    xf = x.reshape(_NW, n_rows // (_NW * _CHUNK), _CHUNK).astype(jnp.int32)
    scale = float(math.sqrt(d_model))
    out = _build(n_rows, vocab, d_model, scale)(xf, table)
    return out.reshape(x.shape + (d_model,))
